# baseline (speedup 1.0000x reference)
import functools

import jax
import jax.numpy as jnp
from jax import lax
from jax.experimental import pallas as pl
from jax.experimental.pallas import tpu as pltpu
from jax.experimental.pallas import tpu_sc as plsc

N = 10000
D = 128
K = 16
H = 128

NP = 10240
EP = NP * K
E_REAL = float(N * K)

RB = 32
NBLK = NP // RB
EB = 2048
UB = EB // K

_HI = jax.lax.Precision.HIGHEST
_INF = float("inf")


_BIG = 2 ** 30
_NT = NP // 128
_L = 4


def _knn_body(xrow_ref, xcol_ref, idx_ref, dist_ref):
    bi = pl.program_id(0)
    x0r = xrow_ref[:, 0:1]
    x1r = xrow_ref[:, 1:2]
    sqr = x0r * x0r + x1r * x1r
    row_g = bi * RB + lax.broadcasted_iota(jnp.int32, (RB, 1), 0)
    pad_row = row_g >= N
    lane = lax.broadcasted_iota(jnp.int32, (1, 128), 1)

    b = lambda t: t.astype(jnp.bfloat16).astype(jnp.float32)
    x0rb = b(x0r)
    x1rb = b(x1r)

    inf = jnp.full((RB, 128), _INF, jnp.float32)
    big = jnp.full((RB, 128), _BIG, jnp.int32)

    def tile(t, carry):
        v0, v1, v2, v3, c0, c1, c2, c3 = carry
        x0c = xcol_ref[0:1, pl.ds(t * 128, 128)]
        x1c = xcol_ref[1:2, pl.ds(t * 128, 128)]
        sqc = x0c * x0c + x1c * x1c
        prod = x0rb * b(x0c) + x1rb * b(x1c)
        d = (sqr + sqc) - 2.0 * prod
        ct = t * 128 + lane
        d = jnp.where(ct == row_g, _INF, d)
        dist_ref[:, pl.ds(t * 128, 128)] = d
        ctb = jnp.broadcast_to(ct, (RB, 128))
        b0 = d < v0
        b1 = d < v1
        b2 = d < v2
        b3 = d < v3
        nv3 = jnp.where(b3, jnp.where(b2, v2, d), v3)
        nc3 = jnp.where(b3, jnp.where(b2, c2, ctb), c3)
        nv2 = jnp.where(b2, jnp.where(b1, v1, d), v2)
        nc2 = jnp.where(b2, jnp.where(b1, c1, ctb), c2)
        nv1 = jnp.where(b1, jnp.where(b0, v0, d), v1)
        nc1 = jnp.where(b1, jnp.where(b0, c0, ctb), c1)
        nv0 = jnp.where(b0, d, v0)
        nc0 = jnp.where(b0, ctb, c0)
        return nv0, nv1, nv2, nv3, nc0, nc1, nc2, nc3

    lv = [inf] * _L
    lc = [big] * _L
    v0, v1, v2, v3, c0, c1, c2, c3 = lax.fori_loop(
        0, _NT, tile, (lv[0], lv[1], lv[2], lv[3], lc[0], lc[1], lc[2], lc[3]))
    lv = [v0, v1, v2, v3]
    lc = [c0, c1, c2, c3]

    cnt = jnp.zeros((RB, 128), jnp.int32)
    for k in range(K):
        vacc = jnp.minimum(jnp.minimum(lv[0], lv[1]),
                           jnp.minimum(lv[2], lv[3]))
        m = jnp.min(vacc, axis=1, keepdims=True)
        cand = big
        for s in range(_L):
            cand = jnp.minimum(cand, jnp.where(lv[s] == m, lc[s], _BIG))
        j = jnp.min(cand, axis=1, keepdims=True)
        idx_ref[:, k:k + 1] = jnp.where(pad_row, jnp.int32(N), j)
        for s in range(_L):
            rm = (lv[s] == m) & (lc[s] == j)
            lv[s] = jnp.where(rm, _INF, lv[s])
        cnt = cnt + jnp.where(lane == jnp.bitwise_and(j, 127), 1, 0)

    bad = jnp.any((cnt >= _L) & jnp.logical_not(pad_row))

    @pl.when(bad)
    def _():
        mprev = jnp.full((RB, 1), -_INF, jnp.float32)
        jprev = jnp.full((RB, 1), -1, jnp.int32)
        for k in range(K):
            def scan(t, carry):
                mv, mc = carry
                d = dist_ref[:, pl.ds(t * 128, 128)]
                ct = jnp.broadcast_to(t * 128 + lane, (RB, 128))
                valid = (d > mprev) | ((d == mprev) & (ct > jprev))
                dm = jnp.where(valid, d, _INF)
                take = dm < mv
                return jnp.where(take, dm, mv), jnp.where(take, ct, mc)

            mv, mc = lax.fori_loop(0, _NT, scan, (inf, big))
            m = jnp.min(mv, axis=1, keepdims=True)
            j = jnp.min(jnp.where(mv == m, mc, _BIG), axis=1, keepdims=True)
            idx_ref[:, k:k + 1] = jnp.where(pad_row, jnp.int32(N), j)
            mprev, jprev = m, j


def _knn(pos_pad, posT):
    return pl.pallas_call(
        _knn_body,
        grid=(NBLK,),
        in_specs=[
            pl.BlockSpec((RB, 2), lambda i: (i, 0)),
            pl.BlockSpec((2, NP), lambda i: (0, 0)),
        ],
        out_specs=pl.BlockSpec((RB, K), lambda i: (i, 0)),
        out_shape=jax.ShapeDtypeStruct((NP, K), jnp.int32),
        scratch_shapes=[pltpu.VMEM((RB, NP), jnp.float32)],
    )(pos_pad, posT)


_UVB = 1024

def _uv_body(x_ref, w_ref, b_ref, u_ref, v_ref):
    bi = pl.program_id(0)
    h = jnp.dot(x_ref[...], w_ref[...], preferred_element_type=jnp.float32,
                precision=_HI)
    row_g = bi * _UVB + lax.broadcasted_iota(jnp.int32, (_UVB, 1), 0)
    mask = row_g < N
    u_ref[...] = h[:, :H] + jnp.where(mask, b_ref[...], 0.0)
    v_ref[...] = h[:, H:]


def _uv(xp, Wuv, b1r):
    return pl.pallas_call(
        _uv_body,
        grid=(NP // _UVB,),
        in_specs=[
            pl.BlockSpec((_UVB, D), lambda i: (i, 0)),
            pl.BlockSpec((D, 2 * H), lambda i: (0, 0)),
            pl.BlockSpec((1, H), lambda i: (0, 0)),
        ],
        out_specs=[
            pl.BlockSpec((_UVB, H), lambda i: (i, 0)),
            pl.BlockSpec((_UVB, H), lambda i: (i, 0)),
        ],
        out_shape=[
            jax.ShapeDtypeStruct((NP, H), jnp.float32),
            jax.ShapeDtypeStruct((NP, H), jnp.float32),
        ],
    )(xp, Wuv, b1r)


_NC = 2
_NS = 16
_NW = _NC * _NS
_PT = EP // _NW
_CH = 320
_NCH = _PT // _CH


def _sc_gather(v_tab, idx_flat):
    mesh = plsc.VectorSubcoreMesh(core_axis_name="c", subcore_axis_name="s")

    @functools.partial(
        pl.kernel,
        out_type=jax.ShapeDtypeStruct((EP, H), jnp.float32),
        mesh=mesh,
        scratch_types=[
            pltpu.VMEM((_PT,), jnp.int32),
            pltpu.VMEM((_CH, H), jnp.float32),
            pltpu.VMEM((_CH, H), jnp.float32),
            pltpu.SemaphoreType.DMA,
        ],
    )
    def body(v_hbm, idx_hbm, out_hbm, idx_v, buf0, buf1, sem):
        wid = lax.axis_index("s") * _NC + lax.axis_index("c")
        base = wid * _PT
        pltpu.sync_copy(idx_hbm.at[pl.ds(base, _PT)], idx_v)
        bufs = (buf0, buf1)
        pend = pltpu.async_copy(v_hbm.at[idx_v.at[pl.ds(0, _CH)]], buf0, sem)
        for c in range(_NCH):
            pend.wait()
            if c + 1 < _NCH:
                pend = pltpu.async_copy(
                    v_hbm.at[idx_v.at[pl.ds((c + 1) * _CH, _CH)]],
                    bufs[(c + 1) % 2], sem)
            pltpu.sync_copy(bufs[c % 2],
                            out_hbm.at[pl.ds(base + c * _CH, _CH)])

    return body(v_tab, idx_flat)


def _urep(u_blk):
    return jnp.broadcast_to(u_blk[:, None, :], (UB, K, H)).reshape(EB, H)


def _stats1_body(vg_ref, u_ref, out_ref, acc_ref):
    b = pl.program_id(0)

    @pl.when(b == 0)
    def _():
        acc_ref[...] = jnp.zeros((8, H), jnp.float32)

    g = vg_ref[...] + _urep(u_ref[...])
    s = jnp.sum(g, axis=0, keepdims=True)
    ss = jnp.sum(g * g, axis=0, keepdims=True)
    acc_ref[0:1, :] += s
    acc_ref[1:2, :] += ss

    @pl.when(b == pl.num_programs(0) - 1)
    def _():
        out_ref[...] = acc_ref[...]


def _stats1(vg, u):
    return pl.pallas_call(
        _stats1_body,
        grid=(EP // EB,),
        in_specs=[
            pl.BlockSpec((EB, H), lambda i: (i, 0)),
            pl.BlockSpec((UB, H), lambda i: (i, 0)),
        ],
        out_specs=pl.BlockSpec((8, H), lambda i: (0, 0)),
        out_shape=jax.ShapeDtypeStruct((8, H), jnp.float32),
        scratch_shapes=[pltpu.VMEM((8, H), jnp.float32)],
    )(vg, u)


def _layer2_body(vg_ref, u_ref, sc1_ref, sh1_ref, w2_ref, b2_ref,
                 h2p_ref, out_ref, acc_ref):
    b = pl.program_id(0)

    @pl.when(b == 0)
    def _():
        acc_ref[...] = jnp.zeros((8, H), jnp.float32)

    g = vg_ref[...] + _urep(u_ref[...])
    h1 = jnp.maximum(g * sc1_ref[...] + sh1_ref[...], 0.0)
    e_g = b * EB + lax.broadcasted_iota(jnp.int32, (EB, 1), 0)
    mask = e_g < N * K
    h1 = jnp.where(mask, h1, 0.0)
    h2 = jnp.dot(h1, w2_ref[...], preferred_element_type=jnp.float32,
                 precision=_HI)
    h2 = h2 + jnp.where(mask, b2_ref[...], 0.0)
    h2p_ref[...] = h2
    acc_ref[0:1, :] += jnp.sum(h2, axis=0, keepdims=True)
    acc_ref[1:2, :] += jnp.sum(h2 * h2, axis=0, keepdims=True)

    @pl.when(b == pl.num_programs(0) - 1)
    def _():
        out_ref[...] = acc_ref[...]


def _layer2(vg, u, sc1, sh1, W2, b2r):
    return pl.pallas_call(
        _layer2_body,
        grid=(EP // EB,),
        in_specs=[
            pl.BlockSpec((EB, H), lambda i: (i, 0)),
            pl.BlockSpec((UB, H), lambda i: (i, 0)),
            pl.BlockSpec((1, H), lambda i: (0, 0)),
            pl.BlockSpec((1, H), lambda i: (0, 0)),
            pl.BlockSpec((H, H), lambda i: (0, 0)),
            pl.BlockSpec((1, H), lambda i: (0, 0)),
        ],
        out_specs=[
            pl.BlockSpec((EB, H), lambda i: (i, 0)),
            pl.BlockSpec((8, H), lambda i: (0, 0)),
        ],
        out_shape=[
            jax.ShapeDtypeStruct((EP, H), jnp.float32),
            jax.ShapeDtypeStruct((8, H), jnp.float32),
        ],
        scratch_shapes=[pltpu.VMEM((8, H), jnp.float32)],
    )(vg, u, sc1, sh1, W2, b2r)


def _reduce_body(h2p_ref, sc2_ref, sh2_ref, out_ref):
    h2 = jnp.maximum(h2p_ref[...] * sc2_ref[...] + sh2_ref[...], 0.0)
    r3 = h2.reshape(UB, K, H)
    acc = r3[:, 0, :]
    for t in range(1, K):
        acc = acc + r3[:, t, :]
    out_ref[...] = acc * (1.0 / K)


def _reduce(h2p, sc2, sh2):
    return pl.pallas_call(
        _reduce_body,
        grid=(EP // EB,),
        in_specs=[
            pl.BlockSpec((EB, H), lambda i: (i, 0)),
            pl.BlockSpec((1, H), lambda i: (0, 0)),
            pl.BlockSpec((1, H), lambda i: (0, 0)),
        ],
        out_specs=pl.BlockSpec((UB, H), lambda i: (i, 0)),
        out_shape=jax.ShapeDtypeStruct((NP, H), jnp.float32),
    )(h2p, sc2, sh2)


@jax.jit
def kernel(x, W1, b1, g1, be1, W2, b2, g2, be2):
    xp = jnp.zeros((NP, D), jnp.float32).at[:N].set(x)
    pos_pad = jnp.full((NP, 2), 1e18, jnp.float32).at[:N].set(x[:, :2])
    posT = pos_pad.T

    idx = _knn(pos_pad, posT)

    Wuv = jnp.concatenate([W1[:D] - W1[D:], W1[D:]], axis=1)
    u, v = _uv(xp, Wuv, b1[None])

    vg = _sc_gather(v, idx.reshape(-1))

    st1 = _stats1(vg, u)
    m1 = st1[0] / E_REAL
    var1 = jnp.maximum(st1[1] / E_REAL - m1 * m1, 0.0)
    sc1 = g1 / jnp.sqrt(var1 + 1e-5)
    sh1 = be1 - m1 * sc1

    h2p, st2 = _layer2(vg, u, sc1[None], sh1[None], W2, b2[None])
    m2 = st2[0] / E_REAL
    var2 = jnp.maximum(st2[1] / E_REAL - m2 * m2, 0.0)
    sc2 = g2 / jnp.sqrt(var2 + 1e-5)
    sh2 = be2 - m2 * sc2

    outp = _reduce(h2p, sc2[None], sh2[None])
    return outp[:N]


# speedup vs baseline: 1.6820x; 1.6820x over previous
import functools

import jax
import jax.numpy as jnp
from jax import lax
from jax.experimental import pallas as pl
from jax.experimental.pallas import tpu as pltpu
from jax.experimental.pallas import tpu_sc as plsc

N = 10000
D = 128
K = 16
H = 128

NP = 10240
EP = NP * K
E_REAL = float(N * K)

RB = 32
NBLK = NP // RB
EB = 2048
UB = EB // K

_HI = jax.lax.Precision.HIGHEST
_INF = float("inf")


_BIG = 2 ** 30
_NT = NP // 128
_L = 4
_TU = 8


def _knn_body(xrow_ref, xcol_ref, idx_ref, dist_ref):
    bi = pl.program_id(0)
    x0r = xrow_ref[:, 0:1]
    x1r = xrow_ref[:, 1:2]
    sqr = x0r * x0r + x1r * x1r
    row_g = bi * RB + lax.broadcasted_iota(jnp.int32, (RB, 1), 0)
    pad_row = row_g >= N
    lane = lax.broadcasted_iota(jnp.int32, (1, 128), 1)

    b = lambda t: t.astype(jnp.bfloat16).astype(jnp.float32)
    x0rb = b(x0r)
    x1rb = b(x1r)

    inf = jnp.full((RB, 128), _INF, jnp.float32)
    big = jnp.full((RB, 128), _BIG, jnp.int32)

    def tile8(it, carry):
        v0, v1, v2, v3, c0, c1, c2, c3 = carry
        for u in range(_TU):
            t = it * _TU + u
            x0c = xcol_ref[0:1, pl.ds(t * 128, 128)]
            x1c = xcol_ref[1:2, pl.ds(t * 128, 128)]
            sqc = x0c * x0c + x1c * x1c
            prod = x0rb * b(x0c) + x1rb * b(x1c)
            d = (sqr + sqc) - 2.0 * prod
            ct = t * 128 + lane
            d = jnp.where(ct == row_g, _INF, d)
            dist_ref[:, pl.ds(t * 128, 128)] = d
            ctb = jnp.broadcast_to(ct, (RB, 128))
            b0 = d < v0
            b1 = d < v1
            b2 = d < v2
            b3 = d < v3
            v3 = jnp.where(b3, jnp.where(b2, v2, d), v3)
            c3 = jnp.where(b3, jnp.where(b2, c2, ctb), c3)
            v2 = jnp.where(b2, jnp.where(b1, v1, d), v2)
            c2 = jnp.where(b2, jnp.where(b1, c1, ctb), c2)
            v1 = jnp.where(b1, jnp.where(b0, v0, d), v1)
            c1 = jnp.where(b1, jnp.where(b0, c0, ctb), c1)
            v0 = jnp.where(b0, d, v0)
            c0 = jnp.where(b0, ctb, c0)
        return v0, v1, v2, v3, c0, c1, c2, c3

    v0, v1, v2, v3, c0, c1, c2, c3 = lax.fori_loop(
        0, _NT // _TU, tile8, (inf, inf, inf, inf, big, big, big, big))
    lv = [v0, v1, v2, v3]
    lc = [c0, c1, c2, c3]

    cnt = jnp.zeros((RB, 128), jnp.int32)
    for k in range(K):
        vacc = jnp.minimum(jnp.minimum(lv[0], lv[1]),
                           jnp.minimum(lv[2], lv[3]))
        m = jnp.min(vacc, axis=1, keepdims=True)
        cand = big
        for s in range(_L):
            cand = jnp.minimum(cand, jnp.where(lv[s] == m, lc[s], _BIG))
        j = jnp.min(cand, axis=1, keepdims=True)
        idx_ref[:, k:k + 1] = jnp.where(pad_row, jnp.int32(N), j)
        for s in range(_L):
            rm = (lv[s] == m) & (lc[s] == j)
            lv[s] = jnp.where(rm, _INF, lv[s])
        cnt = cnt + jnp.where(lane == jnp.bitwise_and(j, 127), 1, 0)

    bad = jnp.any((cnt >= _L) & jnp.logical_not(pad_row))

    @pl.when(bad)
    def _():
        mprev = jnp.full((RB, 1), -_INF, jnp.float32)
        jprev = jnp.full((RB, 1), -1, jnp.int32)
        for k in range(K):
            def scan(t, carry):
                mv, mc = carry
                d = dist_ref[:, pl.ds(t * 128, 128)]
                ct = jnp.broadcast_to(t * 128 + lane, (RB, 128))
                valid = (d > mprev) | ((d == mprev) & (ct > jprev))
                dm = jnp.where(valid, d, _INF)
                take = dm < mv
                return jnp.where(take, dm, mv), jnp.where(take, ct, mc)

            mv, mc = lax.fori_loop(0, _NT, scan, (inf, big))
            m = jnp.min(mv, axis=1, keepdims=True)
            j = jnp.min(jnp.where(mv == m, mc, _BIG), axis=1, keepdims=True)
            idx_ref[:, k:k + 1] = jnp.where(pad_row, jnp.int32(N), j)
            mprev, jprev = m, j


def _knn(pos_pad, posT):
    return pl.pallas_call(
        _knn_body,
        grid=(NBLK,),
        in_specs=[
            pl.BlockSpec((RB, 2), lambda i: (i, 0)),
            pl.BlockSpec((2, NP), lambda i: (0, 0)),
        ],
        out_specs=pl.BlockSpec((RB, K), lambda i: (i, 0)),
        out_shape=jax.ShapeDtypeStruct((NP, K), jnp.int32),
        scratch_shapes=[pltpu.VMEM((RB, NP), jnp.float32)],
    )(pos_pad, posT)


_UVB = 1024

def _uv_body(x_ref, w_ref, b_ref, u_ref, v_ref):
    bi = pl.program_id(0)
    h = jnp.dot(x_ref[...], w_ref[...], preferred_element_type=jnp.float32,
                precision=_HI)
    row_g = bi * _UVB + lax.broadcasted_iota(jnp.int32, (_UVB, 1), 0)
    mask = row_g < N
    u_ref[...] = h[:, :H] + jnp.where(mask, b_ref[...], 0.0)
    v_ref[...] = h[:, H:]


def _uv(xp, Wuv, b1r):
    return pl.pallas_call(
        _uv_body,
        grid=(NP // _UVB,),
        in_specs=[
            pl.BlockSpec((_UVB, D), lambda i: (i, 0)),
            pl.BlockSpec((D, 2 * H), lambda i: (0, 0)),
            pl.BlockSpec((1, H), lambda i: (0, 0)),
        ],
        out_specs=[
            pl.BlockSpec((_UVB, H), lambda i: (i, 0)),
            pl.BlockSpec((_UVB, H), lambda i: (i, 0)),
        ],
        out_shape=[
            jax.ShapeDtypeStruct((NP, H), jnp.float32),
            jax.ShapeDtypeStruct((NP, H), jnp.float32),
        ],
    )(xp, Wuv, b1r)


_NC = 2
_NS = 16
_NW = _NC * _NS
_PT = EP // _NW
_CH = 320
_NCH = _PT // _CH


def _sc_gather(v_tab, idx_flat):
    mesh = plsc.VectorSubcoreMesh(core_axis_name="c", subcore_axis_name="s")

    @functools.partial(
        pl.kernel,
        out_type=jax.ShapeDtypeStruct((EP, H), jnp.float32),
        mesh=mesh,
        scratch_types=[
            pltpu.VMEM((_PT,), jnp.int32),
            pltpu.VMEM((_CH, H), jnp.float32),
            pltpu.VMEM((_CH, H), jnp.float32),
            pltpu.SemaphoreType.DMA,
        ],
    )
    def body(v_hbm, idx_hbm, out_hbm, idx_v, buf0, buf1, sem):
        wid = lax.axis_index("s") * _NC + lax.axis_index("c")
        base = wid * _PT
        pltpu.sync_copy(idx_hbm.at[pl.ds(base, _PT)], idx_v)
        bufs = (buf0, buf1)
        pend = pltpu.async_copy(v_hbm.at[idx_v.at[pl.ds(0, _CH)]], buf0, sem)
        for c in range(_NCH):
            pend.wait()
            if c + 1 < _NCH:
                pend = pltpu.async_copy(
                    v_hbm.at[idx_v.at[pl.ds((c + 1) * _CH, _CH)]],
                    bufs[(c + 1) % 2], sem)
            pltpu.sync_copy(bufs[c % 2],
                            out_hbm.at[pl.ds(base + c * _CH, _CH)])

    return body(v_tab, idx_flat)


def _urep(u_blk):
    return jnp.broadcast_to(u_blk[:, None, :], (UB, K, H)).reshape(EB, H)


def _stats1_body(vg_ref, u_ref, out_ref, acc_ref):
    b = pl.program_id(0)

    @pl.when(b == 0)
    def _():
        acc_ref[...] = jnp.zeros((8, H), jnp.float32)

    g = vg_ref[...] + _urep(u_ref[...])
    s = jnp.sum(g, axis=0, keepdims=True)
    ss = jnp.sum(g * g, axis=0, keepdims=True)
    acc_ref[0:1, :] += s
    acc_ref[1:2, :] += ss

    @pl.when(b == pl.num_programs(0) - 1)
    def _():
        out_ref[...] = acc_ref[...]


def _stats1(vg, u):
    return pl.pallas_call(
        _stats1_body,
        grid=(EP // EB,),
        in_specs=[
            pl.BlockSpec((EB, H), lambda i: (i, 0)),
            pl.BlockSpec((UB, H), lambda i: (i, 0)),
        ],
        out_specs=pl.BlockSpec((8, H), lambda i: (0, 0)),
        out_shape=jax.ShapeDtypeStruct((8, H), jnp.float32),
        scratch_shapes=[pltpu.VMEM((8, H), jnp.float32)],
    )(vg, u)


def _layer2_body(vg_ref, u_ref, sc1_ref, sh1_ref, w2_ref, b2_ref,
                 h2p_ref, out_ref, acc_ref):
    b = pl.program_id(0)

    @pl.when(b == 0)
    def _():
        acc_ref[...] = jnp.zeros((8, H), jnp.float32)

    g = vg_ref[...] + _urep(u_ref[...])
    h1 = jnp.maximum(g * sc1_ref[...] + sh1_ref[...], 0.0)
    e_g = b * EB + lax.broadcasted_iota(jnp.int32, (EB, 1), 0)
    mask = e_g < N * K
    h1 = jnp.where(mask, h1, 0.0)
    h2 = jnp.dot(h1, w2_ref[...], preferred_element_type=jnp.float32,
                 precision=_HI)
    h2 = h2 + jnp.where(mask, b2_ref[...], 0.0)
    h2p_ref[...] = h2
    acc_ref[0:1, :] += jnp.sum(h2, axis=0, keepdims=True)
    acc_ref[1:2, :] += jnp.sum(h2 * h2, axis=0, keepdims=True)

    @pl.when(b == pl.num_programs(0) - 1)
    def _():
        out_ref[...] = acc_ref[...]


def _layer2(vg, u, sc1, sh1, W2, b2r):
    return pl.pallas_call(
        _layer2_body,
        grid=(EP // EB,),
        in_specs=[
            pl.BlockSpec((EB, H), lambda i: (i, 0)),
            pl.BlockSpec((UB, H), lambda i: (i, 0)),
            pl.BlockSpec((1, H), lambda i: (0, 0)),
            pl.BlockSpec((1, H), lambda i: (0, 0)),
            pl.BlockSpec((H, H), lambda i: (0, 0)),
            pl.BlockSpec((1, H), lambda i: (0, 0)),
        ],
        out_specs=[
            pl.BlockSpec((EB, H), lambda i: (i, 0)),
            pl.BlockSpec((8, H), lambda i: (0, 0)),
        ],
        out_shape=[
            jax.ShapeDtypeStruct((EP, H), jnp.float32),
            jax.ShapeDtypeStruct((8, H), jnp.float32),
        ],
        scratch_shapes=[pltpu.VMEM((8, H), jnp.float32)],
    )(vg, u, sc1, sh1, W2, b2r)


def _reduce_body(h2p_ref, sc2_ref, sh2_ref, out_ref):
    h2 = jnp.maximum(h2p_ref[...] * sc2_ref[...] + sh2_ref[...], 0.0)
    r3 = h2.reshape(UB, K, H)
    acc = r3[:, 0, :]
    for t in range(1, K):
        acc = acc + r3[:, t, :]
    out_ref[...] = acc * (1.0 / K)


def _reduce(h2p, sc2, sh2):
    return pl.pallas_call(
        _reduce_body,
        grid=(EP // EB,),
        in_specs=[
            pl.BlockSpec((EB, H), lambda i: (i, 0)),
            pl.BlockSpec((1, H), lambda i: (0, 0)),
            pl.BlockSpec((1, H), lambda i: (0, 0)),
        ],
        out_specs=pl.BlockSpec((UB, H), lambda i: (i, 0)),
        out_shape=jax.ShapeDtypeStruct((NP, H), jnp.float32),
    )(h2p, sc2, sh2)


@jax.jit
def kernel(x, W1, b1, g1, be1, W2, b2, g2, be2):
    xp = jnp.zeros((NP, D), jnp.float32).at[:N].set(x)
    pos_pad = jnp.full((NP, 2), 1e18, jnp.float32).at[:N].set(x[:, :2])
    posT = pos_pad.T

    idx = _knn(pos_pad, posT)

    Wuv = jnp.concatenate([W1[:D] - W1[D:], W1[D:]], axis=1)
    u, v = _uv(xp, Wuv, b1[None])

    vg = _sc_gather(v, idx.reshape(-1))

    st1 = _stats1(vg, u)
    m1 = st1[0] / E_REAL
    var1 = jnp.maximum(st1[1] / E_REAL - m1 * m1, 0.0)
    sc1 = g1 / jnp.sqrt(var1 + 1e-5)
    sh1 = be1 - m1 * sc1

    h2p, st2 = _layer2(vg, u, sc1[None], sh1[None], W2, b2[None])
    m2 = st2[0] / E_REAL
    var2 = jnp.maximum(st2[1] / E_REAL - m2 * m2, 0.0)
    sc2 = g2 / jnp.sqrt(var2 + 1e-5)
    sh2 = be2 - m2 * sc2

    outp = _reduce(h2p, sc2[None], sh2[None])
    return outp[:N]


# speedup vs baseline: 2.0935x; 1.2446x over previous
import functools

import jax
import jax.numpy as jnp
from jax import lax
from jax.experimental import pallas as pl
from jax.experimental.pallas import tpu as pltpu
from jax.experimental.pallas import tpu_sc as plsc

N = 10000
D = 128
K = 16
H = 128

NP = 10240
EP = NP * K
E_REAL = float(N * K)

RB = 128
RG = 32
NG = RB // RG
NBLK = NP // RB
EB = 2048
UB = EB // K

_HI = jax.lax.Precision.HIGHEST
_INF = float("inf")


_BIG = 2 ** 30
_NT = NP // 128
_L = 4
_TU = 8


def _knn_body(xrow_ref, xcol_ref, idx_ref, dist_ref, lv_ref, lc_ref):
    bi = pl.program_id(0)
    row_g = bi * RB + lax.broadcasted_iota(jnp.int32, (RB, 1), 0)
    pad_row = row_g >= N
    lane = lax.broadcasted_iota(jnp.int32, (1, 128), 1)

    b = lambda t: t.astype(jnp.bfloat16).astype(jnp.float32)

    inf = jnp.full((RG, 128), _INF, jnp.float32)
    big = jnp.full((RG, 128), _BIG, jnp.int32)

    for g in range(NG):
        x0r = xrow_ref[g * RG:(g + 1) * RG, 0:1]
        x1r = xrow_ref[g * RG:(g + 1) * RG, 1:2]
        sqr = x0r * x0r + x1r * x1r
        x0rb = b(x0r)
        x1rb = b(x1r)
        rg_g = row_g[g * RG:(g + 1) * RG, :]

        def tile8(it, carry):
            v0, v1, v2, v3, c0, c1, c2, c3 = carry
            for u in range(_TU):
                t = it * _TU + u
                x0c = xcol_ref[0:1, pl.ds(t * 128, 128)]
                x1c = xcol_ref[1:2, pl.ds(t * 128, 128)]
                sqc = x0c * x0c + x1c * x1c
                prod = x0rb * b(x0c) + x1rb * b(x1c)
                d = (sqr + sqc) - 2.0 * prod
                ct = t * 128 + lane
                d = jnp.where(ct == rg_g, _INF, d)
                dist_ref[g * RG:(g + 1) * RG, pl.ds(t * 128, 128)] = d
                ctb = jnp.broadcast_to(ct, (RG, 128))
                b0 = d < v0
                b1 = d < v1
                b2 = d < v2
                b3 = d < v3
                v3 = jnp.where(b3, jnp.where(b2, v2, d), v3)
                c3 = jnp.where(b3, jnp.where(b2, c2, ctb), c3)
                v2 = jnp.where(b2, jnp.where(b1, v1, d), v2)
                c2 = jnp.where(b2, jnp.where(b1, c1, ctb), c2)
                v1 = jnp.where(b1, jnp.where(b0, v0, d), v1)
                c1 = jnp.where(b1, jnp.where(b0, c0, ctb), c1)
                v0 = jnp.where(b0, d, v0)
                c0 = jnp.where(b0, ctb, c0)
            return v0, v1, v2, v3, c0, c1, c2, c3

        res = lax.fori_loop(0, _NT // _TU, tile8,
                            (inf, inf, inf, inf, big, big, big, big))
        for s in range(_L):
            lv_ref[g * RG:(g + 1) * RG, s * 128:(s + 1) * 128] = res[s]
            lc_ref[g * RG:(g + 1) * RG, s * 128:(s + 1) * 128] = res[_L + s]

    cnt = jnp.zeros((RB, 128), jnp.int32)
    for k in range(K):
        d = lv_ref[...]
        c = lc_ref[...]
        m = jnp.min(d, axis=1, keepdims=True)
        j = jnp.min(jnp.where(d == m, c, _BIG), axis=1, keepdims=True)
        idx_ref[:, k:k + 1] = jnp.where(pad_row, jnp.int32(N), j)
        lv_ref[...] = jnp.where((d == m) & (c == j), _INF, d)
        cnt = cnt + jnp.where(lane == jnp.bitwise_and(j, 127), 1, 0)

    bad = jnp.any((cnt >= _L) & jnp.logical_not(pad_row))

    @pl.when(bad)
    def _():
        infb = jnp.full((RB, 128), _INF, jnp.float32)
        bigb = jnp.full((RB, 128), _BIG, jnp.int32)
        mprev = jnp.full((RB, 1), -_INF, jnp.float32)
        jprev = jnp.full((RB, 1), -1, jnp.int32)
        for k in range(K):
            def scan(t, carry):
                mv, mc = carry
                d = dist_ref[:, pl.ds(t * 128, 128)]
                ct = jnp.broadcast_to(t * 128 + lane, (RB, 128))
                valid = (d > mprev) | ((d == mprev) & (ct > jprev))
                dm = jnp.where(valid, d, _INF)
                take = dm < mv
                return jnp.where(take, dm, mv), jnp.where(take, ct, mc)

            mv, mc = lax.fori_loop(0, _NT, scan, (infb, bigb))
            m = jnp.min(mv, axis=1, keepdims=True)
            j = jnp.min(jnp.where(mv == m, mc, _BIG), axis=1, keepdims=True)
            idx_ref[:, k:k + 1] = jnp.where(pad_row, jnp.int32(N), j)
            mprev, jprev = m, j


def _knn(pos_pad, posT):
    return pl.pallas_call(
        _knn_body,
        grid=(NBLK,),
        in_specs=[
            pl.BlockSpec((RB, 2), lambda i: (i, 0)),
            pl.BlockSpec((2, NP), lambda i: (0, 0)),
        ],
        out_specs=pl.BlockSpec((RB, K), lambda i: (i, 0)),
        out_shape=jax.ShapeDtypeStruct((NP, K), jnp.int32),
        scratch_shapes=[
            pltpu.VMEM((RB, NP), jnp.float32),
            pltpu.VMEM((RB, _L * 128), jnp.float32),
            pltpu.VMEM((RB, _L * 128), jnp.int32),
        ],
    )(pos_pad, posT)


_UVB = 1024

def _uv_body(x_ref, w_ref, b_ref, u_ref, v_ref):
    bi = pl.program_id(0)
    h = jnp.dot(x_ref[...], w_ref[...], preferred_element_type=jnp.float32,
                precision=_HI)
    row_g = bi * _UVB + lax.broadcasted_iota(jnp.int32, (_UVB, 1), 0)
    mask = row_g < N
    u_ref[...] = h[:, :H] + jnp.where(mask, b_ref[...], 0.0)
    v_ref[...] = h[:, H:]


def _uv(xp, Wuv, b1r):
    return pl.pallas_call(
        _uv_body,
        grid=(NP // _UVB,),
        in_specs=[
            pl.BlockSpec((_UVB, D), lambda i: (i, 0)),
            pl.BlockSpec((D, 2 * H), lambda i: (0, 0)),
            pl.BlockSpec((1, H), lambda i: (0, 0)),
        ],
        out_specs=[
            pl.BlockSpec((_UVB, H), lambda i: (i, 0)),
            pl.BlockSpec((_UVB, H), lambda i: (i, 0)),
        ],
        out_shape=[
            jax.ShapeDtypeStruct((NP, H), jnp.float32),
            jax.ShapeDtypeStruct((NP, H), jnp.float32),
        ],
    )(xp, Wuv, b1r)


_NC = 2
_NS = 16
_NW = _NC * _NS
_PT = EP // _NW
_CH = 320
_NCH = _PT // _CH


def _sc_gather(v_tab, idx_flat):
    mesh = plsc.VectorSubcoreMesh(core_axis_name="c", subcore_axis_name="s")

    @functools.partial(
        pl.kernel,
        out_type=jax.ShapeDtypeStruct((EP, H), jnp.float32),
        mesh=mesh,
        scratch_types=[
            pltpu.VMEM((_PT,), jnp.int32),
            pltpu.VMEM((_CH, H), jnp.float32),
            pltpu.VMEM((_CH, H), jnp.float32),
            pltpu.SemaphoreType.DMA,
        ],
    )
    def body(v_hbm, idx_hbm, out_hbm, idx_v, buf0, buf1, sem):
        wid = lax.axis_index("s") * _NC + lax.axis_index("c")
        base = wid * _PT
        pltpu.sync_copy(idx_hbm.at[pl.ds(base, _PT)], idx_v)
        bufs = (buf0, buf1)
        pend = pltpu.async_copy(v_hbm.at[idx_v.at[pl.ds(0, _CH)]], buf0, sem)
        for c in range(_NCH):
            pend.wait()
            if c + 1 < _NCH:
                pend = pltpu.async_copy(
                    v_hbm.at[idx_v.at[pl.ds((c + 1) * _CH, _CH)]],
                    bufs[(c + 1) % 2], sem)
            pltpu.sync_copy(bufs[c % 2],
                            out_hbm.at[pl.ds(base + c * _CH, _CH)])

    return body(v_tab, idx_flat)


def _urep(u_blk):
    return jnp.broadcast_to(u_blk[:, None, :], (UB, K, H)).reshape(EB, H)


def _stats1_body(vg_ref, u_ref, out_ref, acc_ref):
    b = pl.program_id(0)

    @pl.when(b == 0)
    def _():
        acc_ref[...] = jnp.zeros((8, H), jnp.float32)

    g = vg_ref[...] + _urep(u_ref[...])
    s = jnp.sum(g, axis=0, keepdims=True)
    ss = jnp.sum(g * g, axis=0, keepdims=True)
    acc_ref[0:1, :] += s
    acc_ref[1:2, :] += ss

    @pl.when(b == pl.num_programs(0) - 1)
    def _():
        out_ref[...] = acc_ref[...]


def _stats1(vg, u):
    return pl.pallas_call(
        _stats1_body,
        grid=(EP // EB,),
        in_specs=[
            pl.BlockSpec((EB, H), lambda i: (i, 0)),
            pl.BlockSpec((UB, H), lambda i: (i, 0)),
        ],
        out_specs=pl.BlockSpec((8, H), lambda i: (0, 0)),
        out_shape=jax.ShapeDtypeStruct((8, H), jnp.float32),
        scratch_shapes=[pltpu.VMEM((8, H), jnp.float32)],
    )(vg, u)


def _layer2_body(vg_ref, u_ref, sc1_ref, sh1_ref, w2_ref, b2_ref,
                 h2p_ref, out_ref, acc_ref):
    b = pl.program_id(0)

    @pl.when(b == 0)
    def _():
        acc_ref[...] = jnp.zeros((8, H), jnp.float32)

    g = vg_ref[...] + _urep(u_ref[...])
    h1 = jnp.maximum(g * sc1_ref[...] + sh1_ref[...], 0.0)
    e_g = b * EB + lax.broadcasted_iota(jnp.int32, (EB, 1), 0)
    mask = e_g < N * K
    h1 = jnp.where(mask, h1, 0.0)
    h2 = jnp.dot(h1, w2_ref[...], preferred_element_type=jnp.float32,
                 precision=_HI)
    h2 = h2 + jnp.where(mask, b2_ref[...], 0.0)
    h2p_ref[...] = h2
    acc_ref[0:1, :] += jnp.sum(h2, axis=0, keepdims=True)
    acc_ref[1:2, :] += jnp.sum(h2 * h2, axis=0, keepdims=True)

    @pl.when(b == pl.num_programs(0) - 1)
    def _():
        out_ref[...] = acc_ref[...]


def _layer2(vg, u, sc1, sh1, W2, b2r):
    return pl.pallas_call(
        _layer2_body,
        grid=(EP // EB,),
        in_specs=[
            pl.BlockSpec((EB, H), lambda i: (i, 0)),
            pl.BlockSpec((UB, H), lambda i: (i, 0)),
            pl.BlockSpec((1, H), lambda i: (0, 0)),
            pl.BlockSpec((1, H), lambda i: (0, 0)),
            pl.BlockSpec((H, H), lambda i: (0, 0)),
            pl.BlockSpec((1, H), lambda i: (0, 0)),
        ],
        out_specs=[
            pl.BlockSpec((EB, H), lambda i: (i, 0)),
            pl.BlockSpec((8, H), lambda i: (0, 0)),
        ],
        out_shape=[
            jax.ShapeDtypeStruct((EP, H), jnp.float32),
            jax.ShapeDtypeStruct((8, H), jnp.float32),
        ],
        scratch_shapes=[pltpu.VMEM((8, H), jnp.float32)],
    )(vg, u, sc1, sh1, W2, b2r)


def _reduce_body(h2p_ref, sc2_ref, sh2_ref, out_ref):
    h2 = jnp.maximum(h2p_ref[...] * sc2_ref[...] + sh2_ref[...], 0.0)
    r3 = h2.reshape(UB, K, H)
    acc = r3[:, 0, :]
    for t in range(1, K):
        acc = acc + r3[:, t, :]
    out_ref[...] = acc * (1.0 / K)


def _reduce(h2p, sc2, sh2):
    return pl.pallas_call(
        _reduce_body,
        grid=(EP // EB,),
        in_specs=[
            pl.BlockSpec((EB, H), lambda i: (i, 0)),
            pl.BlockSpec((1, H), lambda i: (0, 0)),
            pl.BlockSpec((1, H), lambda i: (0, 0)),
        ],
        out_specs=pl.BlockSpec((UB, H), lambda i: (i, 0)),
        out_shape=jax.ShapeDtypeStruct((NP, H), jnp.float32),
    )(h2p, sc2, sh2)


@jax.jit
def kernel(x, W1, b1, g1, be1, W2, b2, g2, be2):
    xp = jnp.zeros((NP, D), jnp.float32).at[:N].set(x)
    pos_pad = jnp.full((NP, 2), 1e18, jnp.float32).at[:N].set(x[:, :2])
    posT = pos_pad.T

    idx = _knn(pos_pad, posT)

    Wuv = jnp.concatenate([W1[:D] - W1[D:], W1[D:]], axis=1)
    u, v = _uv(xp, Wuv, b1[None])

    vg = _sc_gather(v, idx.reshape(-1))

    st1 = _stats1(vg, u)
    m1 = st1[0] / E_REAL
    var1 = jnp.maximum(st1[1] / E_REAL - m1 * m1, 0.0)
    sc1 = g1 / jnp.sqrt(var1 + 1e-5)
    sh1 = be1 - m1 * sc1

    h2p, st2 = _layer2(vg, u, sc1[None], sh1[None], W2, b2[None])
    m2 = st2[0] / E_REAL
    var2 = jnp.maximum(st2[1] / E_REAL - m2 * m2, 0.0)
    sc2 = g2 / jnp.sqrt(var2 + 1e-5)
    sh2 = be2 - m2 * sc2

    outp = _reduce(h2p, sc2[None], sh2[None])
    return outp[:N]


# speedup vs baseline: 2.2343x; 1.0672x over previous
import functools

import jax
import jax.numpy as jnp
from jax import lax
from jax.experimental import pallas as pl
from jax.experimental.pallas import tpu as pltpu
from jax.experimental.pallas import tpu_sc as plsc

N = 10000
D = 128
K = 16
H = 128

NP = 10240
EP = NP * K
E_REAL = float(N * K)

RB = 128
RG = 32
NG = RB // RG
NBLK = NP // RB
EB = 2048
UB = EB // K

_HI = jax.lax.Precision.HIGHEST
_INF = float("inf")


_BIG = 2 ** 30
_NT = NP // 128
_L = 4
_TU = 8


def _knn_body(xrow_ref, xcol_ref, idx_ref, dist_ref, lv_ref, lc_ref):
    bi = pl.program_id(0)
    row_g = bi * RB + lax.broadcasted_iota(jnp.int32, (RB, 1), 0)
    pad_row = row_g >= N
    lane = lax.broadcasted_iota(jnp.int32, (1, 128), 1)

    b = lambda t: t.astype(jnp.bfloat16).astype(jnp.float32)

    lv_ref[...] = jnp.full((RB, _L * 128), _INF, jnp.float32)
    lc_ref[...] = jnp.full((RB, _L * 128), _BIG, jnp.int32)

    def step(it, _u):
        for g in range(NG):
            rsl = slice(g * RG, (g + 1) * RG)
            x0r = xrow_ref[rsl, 0:1]
            x1r = xrow_ref[rsl, 1:2]
            sqr = x0r * x0r + x1r * x1r
            x0rb = b(x0r)
            x1rb = b(x1r)
            rg_g = row_g[rsl, :]
            v0 = lv_ref[rsl, 0:128]
            v1 = lv_ref[rsl, 128:256]
            v2 = lv_ref[rsl, 256:384]
            v3 = lv_ref[rsl, 384:512]
            c0 = lc_ref[rsl, 0:128]
            c1 = lc_ref[rsl, 128:256]
            c2 = lc_ref[rsl, 256:384]
            c3 = lc_ref[rsl, 384:512]
            for u in range(_TU):
                t = it * _TU + u
                x0c = xcol_ref[0:1, pl.ds(t * 128, 128)]
                x1c = xcol_ref[1:2, pl.ds(t * 128, 128)]
                sqc = x0c * x0c + x1c * x1c
                prod = x0rb * b(x0c) + x1rb * b(x1c)
                d = (sqr + sqc) - 2.0 * prod
                ct = t * 128 + lane
                d = jnp.where(ct == rg_g, _INF, d)
                dist_ref[rsl, pl.ds(t * 128, 128)] = d
                ctb = jnp.broadcast_to(ct, (RG, 128))
                b0 = d < v0
                b1 = d < v1
                b2 = d < v2
                b3 = d < v3
                v3 = jnp.where(b3, jnp.where(b2, v2, d), v3)
                c3 = jnp.where(b3, jnp.where(b2, c2, ctb), c3)
                v2 = jnp.where(b2, jnp.where(b1, v1, d), v2)
                c2 = jnp.where(b2, jnp.where(b1, c1, ctb), c2)
                v1 = jnp.where(b1, jnp.where(b0, v0, d), v1)
                c1 = jnp.where(b1, jnp.where(b0, c0, ctb), c1)
                v0 = jnp.where(b0, d, v0)
                c0 = jnp.where(b0, ctb, c0)
            lv_ref[rsl, 0:128] = v0
            lv_ref[rsl, 128:256] = v1
            lv_ref[rsl, 256:384] = v2
            lv_ref[rsl, 384:512] = v3
            lc_ref[rsl, 0:128] = c0
            lc_ref[rsl, 128:256] = c1
            lc_ref[rsl, 256:384] = c2
            lc_ref[rsl, 384:512] = c3
        return _u

    lax.fori_loop(0, _NT // _TU, step, 0)

    cnt = jnp.zeros((RB, 128), jnp.int32)
    for k in range(K):
        ls = [lv_ref[:, s * 128:(s + 1) * 128] for s in range(_L)]
        cs = [lc_ref[:, s * 128:(s + 1) * 128] for s in range(_L)]
        vacc = jnp.minimum(jnp.minimum(ls[0], ls[1]),
                           jnp.minimum(ls[2], ls[3]))
        m = jnp.min(vacc, axis=1, keepdims=True)
        cand = jnp.full((RB, 128), _BIG, jnp.int32)
        for s in range(_L):
            cand = jnp.minimum(cand, jnp.where(ls[s] == m, cs[s], _BIG))
        j = jnp.min(cand, axis=1, keepdims=True)
        idx_ref[:, k:k + 1] = jnp.where(pad_row, jnp.int32(N), j)
        for s in range(_L):
            lv_ref[:, s * 128:(s + 1) * 128] = jnp.where(
                (ls[s] == m) & (cs[s] == j), _INF, ls[s])
        cnt = cnt + jnp.where(lane == jnp.bitwise_and(j, 127), 1, 0)

    bad = jnp.any((cnt >= _L) & jnp.logical_not(pad_row))

    @pl.when(bad)
    def _():
        infb = jnp.full((RB, 128), _INF, jnp.float32)
        bigb = jnp.full((RB, 128), _BIG, jnp.int32)
        mprev = jnp.full((RB, 1), -_INF, jnp.float32)
        jprev = jnp.full((RB, 1), -1, jnp.int32)
        for k in range(K):
            def scan(t, carry):
                mv, mc = carry
                d = dist_ref[:, pl.ds(t * 128, 128)]
                ct = jnp.broadcast_to(t * 128 + lane, (RB, 128))
                valid = (d > mprev) | ((d == mprev) & (ct > jprev))
                dm = jnp.where(valid, d, _INF)
                take = dm < mv
                return jnp.where(take, dm, mv), jnp.where(take, ct, mc)

            mv, mc = lax.fori_loop(0, _NT, scan, (infb, bigb))
            m = jnp.min(mv, axis=1, keepdims=True)
            j = jnp.min(jnp.where(mv == m, mc, _BIG), axis=1, keepdims=True)
            idx_ref[:, k:k + 1] = jnp.where(pad_row, jnp.int32(N), j)
            mprev, jprev = m, j


def _knn(pos_pad, posT):
    return pl.pallas_call(
        _knn_body,
        grid=(NBLK,),
        in_specs=[
            pl.BlockSpec((RB, 2), lambda i: (i, 0)),
            pl.BlockSpec((2, NP), lambda i: (0, 0)),
        ],
        out_specs=pl.BlockSpec((RB, K), lambda i: (i, 0)),
        out_shape=jax.ShapeDtypeStruct((NP, K), jnp.int32),
        scratch_shapes=[
            pltpu.VMEM((RB, NP), jnp.float32),
            pltpu.VMEM((RB, _L * 128), jnp.float32),
            pltpu.VMEM((RB, _L * 128), jnp.int32),
        ],
    )(pos_pad, posT)


_UVB = 1024

def _uv_body(x_ref, w_ref, b_ref, u_ref, v_ref):
    bi = pl.program_id(0)
    h = jnp.dot(x_ref[...], w_ref[...], preferred_element_type=jnp.float32,
                precision=_HI)
    row_g = bi * _UVB + lax.broadcasted_iota(jnp.int32, (_UVB, 1), 0)
    mask = row_g < N
    u_ref[...] = h[:, :H] + jnp.where(mask, b_ref[...], 0.0)
    v_ref[...] = h[:, H:]


def _uv(xp, Wuv, b1r):
    return pl.pallas_call(
        _uv_body,
        grid=(NP // _UVB,),
        in_specs=[
            pl.BlockSpec((_UVB, D), lambda i: (i, 0)),
            pl.BlockSpec((D, 2 * H), lambda i: (0, 0)),
            pl.BlockSpec((1, H), lambda i: (0, 0)),
        ],
        out_specs=[
            pl.BlockSpec((_UVB, H), lambda i: (i, 0)),
            pl.BlockSpec((_UVB, H), lambda i: (i, 0)),
        ],
        out_shape=[
            jax.ShapeDtypeStruct((NP, H), jnp.float32),
            jax.ShapeDtypeStruct((NP, H), jnp.float32),
        ],
    )(xp, Wuv, b1r)


_NC = 2
_NS = 16
_NW = _NC * _NS
_PT = EP // _NW
_CH = 320
_NCH = _PT // _CH


def _sc_gather(v_tab, idx_flat):
    mesh = plsc.VectorSubcoreMesh(core_axis_name="c", subcore_axis_name="s")

    @functools.partial(
        pl.kernel,
        out_type=jax.ShapeDtypeStruct((EP, H), jnp.float32),
        mesh=mesh,
        scratch_types=[
            pltpu.VMEM((_PT,), jnp.int32),
            pltpu.VMEM((_CH, H), jnp.float32),
            pltpu.VMEM((_CH, H), jnp.float32),
            pltpu.SemaphoreType.DMA,
        ],
    )
    def body(v_hbm, idx_hbm, out_hbm, idx_v, buf0, buf1, sem):
        wid = lax.axis_index("s") * _NC + lax.axis_index("c")
        base = wid * _PT
        pltpu.sync_copy(idx_hbm.at[pl.ds(base, _PT)], idx_v)
        bufs = (buf0, buf1)
        pend = pltpu.async_copy(v_hbm.at[idx_v.at[pl.ds(0, _CH)]], buf0, sem)
        for c in range(_NCH):
            pend.wait()
            if c + 1 < _NCH:
                pend = pltpu.async_copy(
                    v_hbm.at[idx_v.at[pl.ds((c + 1) * _CH, _CH)]],
                    bufs[(c + 1) % 2], sem)
            pltpu.sync_copy(bufs[c % 2],
                            out_hbm.at[pl.ds(base + c * _CH, _CH)])

    return body(v_tab, idx_flat)


def _urep(u_blk):
    return jnp.broadcast_to(u_blk[:, None, :], (UB, K, H)).reshape(EB, H)


def _stats1_body(vg_ref, u_ref, out_ref, acc_ref):
    b = pl.program_id(0)

    @pl.when(b == 0)
    def _():
        acc_ref[...] = jnp.zeros((8, H), jnp.float32)

    g = vg_ref[...] + _urep(u_ref[...])
    s = jnp.sum(g, axis=0, keepdims=True)
    ss = jnp.sum(g * g, axis=0, keepdims=True)
    acc_ref[0:1, :] += s
    acc_ref[1:2, :] += ss

    @pl.when(b == pl.num_programs(0) - 1)
    def _():
        out_ref[...] = acc_ref[...]


def _stats1(vg, u):
    return pl.pallas_call(
        _stats1_body,
        grid=(EP // EB,),
        in_specs=[
            pl.BlockSpec((EB, H), lambda i: (i, 0)),
            pl.BlockSpec((UB, H), lambda i: (i, 0)),
        ],
        out_specs=pl.BlockSpec((8, H), lambda i: (0, 0)),
        out_shape=jax.ShapeDtypeStruct((8, H), jnp.float32),
        scratch_shapes=[pltpu.VMEM((8, H), jnp.float32)],
    )(vg, u)


def _layer2_body(vg_ref, u_ref, sc1_ref, sh1_ref, w2_ref, b2_ref,
                 h2p_ref, out_ref, acc_ref):
    b = pl.program_id(0)

    @pl.when(b == 0)
    def _():
        acc_ref[...] = jnp.zeros((8, H), jnp.float32)

    g = vg_ref[...] + _urep(u_ref[...])
    h1 = jnp.maximum(g * sc1_ref[...] + sh1_ref[...], 0.0)
    e_g = b * EB + lax.broadcasted_iota(jnp.int32, (EB, 1), 0)
    mask = e_g < N * K
    h1 = jnp.where(mask, h1, 0.0)
    h2 = jnp.dot(h1, w2_ref[...], preferred_element_type=jnp.float32,
                 precision=_HI)
    h2 = h2 + jnp.where(mask, b2_ref[...], 0.0)
    h2p_ref[...] = h2
    acc_ref[0:1, :] += jnp.sum(h2, axis=0, keepdims=True)
    acc_ref[1:2, :] += jnp.sum(h2 * h2, axis=0, keepdims=True)

    @pl.when(b == pl.num_programs(0) - 1)
    def _():
        out_ref[...] = acc_ref[...]


def _layer2(vg, u, sc1, sh1, W2, b2r):
    return pl.pallas_call(
        _layer2_body,
        grid=(EP // EB,),
        in_specs=[
            pl.BlockSpec((EB, H), lambda i: (i, 0)),
            pl.BlockSpec((UB, H), lambda i: (i, 0)),
            pl.BlockSpec((1, H), lambda i: (0, 0)),
            pl.BlockSpec((1, H), lambda i: (0, 0)),
            pl.BlockSpec((H, H), lambda i: (0, 0)),
            pl.BlockSpec((1, H), lambda i: (0, 0)),
        ],
        out_specs=[
            pl.BlockSpec((EB, H), lambda i: (i, 0)),
            pl.BlockSpec((8, H), lambda i: (0, 0)),
        ],
        out_shape=[
            jax.ShapeDtypeStruct((EP, H), jnp.float32),
            jax.ShapeDtypeStruct((8, H), jnp.float32),
        ],
        scratch_shapes=[pltpu.VMEM((8, H), jnp.float32)],
    )(vg, u, sc1, sh1, W2, b2r)


def _reduce_body(h2p_ref, sc2_ref, sh2_ref, out_ref):
    h2 = jnp.maximum(h2p_ref[...] * sc2_ref[...] + sh2_ref[...], 0.0)
    r3 = h2.reshape(UB, K, H)
    acc = r3[:, 0, :]
    for t in range(1, K):
        acc = acc + r3[:, t, :]
    out_ref[...] = acc * (1.0 / K)


def _reduce(h2p, sc2, sh2):
    return pl.pallas_call(
        _reduce_body,
        grid=(EP // EB,),
        in_specs=[
            pl.BlockSpec((EB, H), lambda i: (i, 0)),
            pl.BlockSpec((1, H), lambda i: (0, 0)),
            pl.BlockSpec((1, H), lambda i: (0, 0)),
        ],
        out_specs=pl.BlockSpec((UB, H), lambda i: (i, 0)),
        out_shape=jax.ShapeDtypeStruct((NP, H), jnp.float32),
    )(h2p, sc2, sh2)


@jax.jit
def kernel(x, W1, b1, g1, be1, W2, b2, g2, be2):
    xp = jnp.zeros((NP, D), jnp.float32).at[:N].set(x)
    pos_pad = jnp.full((NP, 2), 1e18, jnp.float32).at[:N].set(x[:, :2])
    posT = pos_pad.T

    idx = _knn(pos_pad, posT)

    Wuv = jnp.concatenate([W1[:D] - W1[D:], W1[D:]], axis=1)
    u, v = _uv(xp, Wuv, b1[None])

    vg = _sc_gather(v, idx.reshape(-1))

    st1 = _stats1(vg, u)
    m1 = st1[0] / E_REAL
    var1 = jnp.maximum(st1[1] / E_REAL - m1 * m1, 0.0)
    sc1 = g1 / jnp.sqrt(var1 + 1e-5)
    sh1 = be1 - m1 * sc1

    h2p, st2 = _layer2(vg, u, sc1[None], sh1[None], W2, b2[None])
    m2 = st2[0] / E_REAL
    var2 = jnp.maximum(st2[1] / E_REAL - m2 * m2, 0.0)
    sc2 = g2 / jnp.sqrt(var2 + 1e-5)
    sh2 = be2 - m2 * sc2

    outp = _reduce(h2p, sc2[None], sh2[None])
    return outp[:N]


# speedup vs baseline: 2.2617x; 1.0123x over previous
import functools

import jax
import jax.numpy as jnp
from jax import lax
from jax.experimental import pallas as pl
from jax.experimental.pallas import tpu as pltpu
from jax.experimental.pallas import tpu_sc as plsc

N = 10000
D = 128
K = 16
H = 128

NP = 10240
EP = NP * K
E_REAL = float(N * K)

RB = 128
RG = 32
NG = RB // RG
NBLK = NP // RB
EB = 2048
UB = EB // K

_HI = jax.lax.Precision.HIGHEST
_INF = float("inf")


_BIG = 2 ** 30
_NT = NP // 128
_L = 4
_TU = 16


def _knn_body(xrow_ref, xcol_ref, idx_ref, dist_ref, lv_ref, lc_ref):
    bi = pl.program_id(0)
    row_g = bi * RB + lax.broadcasted_iota(jnp.int32, (RB, 1), 0)
    pad_row = row_g >= N
    lane = lax.broadcasted_iota(jnp.int32, (1, 128), 1)

    b = lambda t: t.astype(jnp.bfloat16).astype(jnp.float32)

    lv_ref[...] = jnp.full((RB, _L * 128), _INF, jnp.float32)
    lc_ref[...] = jnp.full((RB, _L * 128), _BIG, jnp.int32)

    def step(it, _u):
        for g in range(NG):
            rsl = slice(g * RG, (g + 1) * RG)
            x0r = xrow_ref[rsl, 0:1]
            x1r = xrow_ref[rsl, 1:2]
            sqr = x0r * x0r + x1r * x1r
            x0rb = b(x0r)
            x1rb = b(x1r)
            rg_g = row_g[rsl, :]
            v0 = lv_ref[rsl, 0:128]
            v1 = lv_ref[rsl, 128:256]
            v2 = lv_ref[rsl, 256:384]
            v3 = lv_ref[rsl, 384:512]
            c0 = lc_ref[rsl, 0:128]
            c1 = lc_ref[rsl, 128:256]
            c2 = lc_ref[rsl, 256:384]
            c3 = lc_ref[rsl, 384:512]
            for u in range(_TU):
                t = it * _TU + u
                x0c = xcol_ref[0:1, pl.ds(t * 128, 128)]
                x1c = xcol_ref[1:2, pl.ds(t * 128, 128)]
                sqc = x0c * x0c + x1c * x1c
                prod = x0rb * b(x0c) + x1rb * b(x1c)
                d = (sqr + sqc) - 2.0 * prod
                ct = t * 128 + lane
                d = jnp.where(ct == rg_g, _INF, d)
                dist_ref[rsl, pl.ds(t * 128, 128)] = d
                ctb = jnp.broadcast_to(ct, (RG, 128))
                b0 = d < v0
                b1 = d < v1
                b2 = d < v2
                b3 = d < v3
                v3 = jnp.where(b3, jnp.where(b2, v2, d), v3)
                c3 = jnp.where(b3, jnp.where(b2, c2, ctb), c3)
                v2 = jnp.where(b2, jnp.where(b1, v1, d), v2)
                c2 = jnp.where(b2, jnp.where(b1, c1, ctb), c2)
                v1 = jnp.where(b1, jnp.where(b0, v0, d), v1)
                c1 = jnp.where(b1, jnp.where(b0, c0, ctb), c1)
                v0 = jnp.where(b0, d, v0)
                c0 = jnp.where(b0, ctb, c0)
            lv_ref[rsl, 0:128] = v0
            lv_ref[rsl, 128:256] = v1
            lv_ref[rsl, 256:384] = v2
            lv_ref[rsl, 384:512] = v3
            lc_ref[rsl, 0:128] = c0
            lc_ref[rsl, 128:256] = c1
            lc_ref[rsl, 256:384] = c2
            lc_ref[rsl, 384:512] = c3
        return _u

    lax.fori_loop(0, _NT // _TU, step, 0)

    cnt = jnp.zeros((RB, 128), jnp.int32)
    for k in range(K):
        ls = [lv_ref[:, s * 128:(s + 1) * 128] for s in range(_L)]
        cs = [lc_ref[:, s * 128:(s + 1) * 128] for s in range(_L)]
        vacc = jnp.minimum(jnp.minimum(ls[0], ls[1]),
                           jnp.minimum(ls[2], ls[3]))
        m = jnp.min(vacc, axis=1, keepdims=True)
        cand = jnp.full((RB, 128), _BIG, jnp.int32)
        for s in range(_L):
            cand = jnp.minimum(cand, jnp.where(ls[s] == m, cs[s], _BIG))
        j = jnp.min(cand, axis=1, keepdims=True)
        idx_ref[:, k:k + 1] = jnp.where(pad_row, jnp.int32(N), j)
        for s in range(_L):
            lv_ref[:, s * 128:(s + 1) * 128] = jnp.where(
                (ls[s] == m) & (cs[s] == j), _INF, ls[s])
        cnt = cnt + jnp.where(lane == jnp.bitwise_and(j, 127), 1, 0)

    bad = jnp.any((cnt >= _L) & jnp.logical_not(pad_row))

    @pl.when(bad)
    def _():
        infb = jnp.full((RB, 128), _INF, jnp.float32)
        bigb = jnp.full((RB, 128), _BIG, jnp.int32)
        mprev = jnp.full((RB, 1), -_INF, jnp.float32)
        jprev = jnp.full((RB, 1), -1, jnp.int32)
        for k in range(K):
            def scan(t, carry):
                mv, mc = carry
                d = dist_ref[:, pl.ds(t * 128, 128)]
                ct = jnp.broadcast_to(t * 128 + lane, (RB, 128))
                valid = (d > mprev) | ((d == mprev) & (ct > jprev))
                dm = jnp.where(valid, d, _INF)
                take = dm < mv
                return jnp.where(take, dm, mv), jnp.where(take, ct, mc)

            mv, mc = lax.fori_loop(0, _NT, scan, (infb, bigb))
            m = jnp.min(mv, axis=1, keepdims=True)
            j = jnp.min(jnp.where(mv == m, mc, _BIG), axis=1, keepdims=True)
            idx_ref[:, k:k + 1] = jnp.where(pad_row, jnp.int32(N), j)
            mprev, jprev = m, j


def _knn(pos_pad, posT):
    return pl.pallas_call(
        _knn_body,
        grid=(NBLK,),
        in_specs=[
            pl.BlockSpec((RB, 2), lambda i: (i, 0)),
            pl.BlockSpec((2, NP), lambda i: (0, 0)),
        ],
        out_specs=pl.BlockSpec((RB, K), lambda i: (i, 0)),
        out_shape=jax.ShapeDtypeStruct((NP, K), jnp.int32),
        scratch_shapes=[
            pltpu.VMEM((RB, NP), jnp.float32),
            pltpu.VMEM((RB, _L * 128), jnp.float32),
            pltpu.VMEM((RB, _L * 128), jnp.int32),
        ],
    )(pos_pad, posT)


_UVB = 1024

def _uv_body(x_ref, w_ref, b_ref, u_ref, v_ref):
    bi = pl.program_id(0)
    h = jnp.dot(x_ref[...], w_ref[...], preferred_element_type=jnp.float32,
                precision=_HI)
    row_g = bi * _UVB + lax.broadcasted_iota(jnp.int32, (_UVB, 1), 0)
    mask = row_g < N
    u_ref[...] = h[:, :H] + jnp.where(mask, b_ref[...], 0.0)
    v_ref[...] = h[:, H:]


def _uv(xp, Wuv, b1r):
    return pl.pallas_call(
        _uv_body,
        grid=(NP // _UVB,),
        in_specs=[
            pl.BlockSpec((_UVB, D), lambda i: (i, 0)),
            pl.BlockSpec((D, 2 * H), lambda i: (0, 0)),
            pl.BlockSpec((1, H), lambda i: (0, 0)),
        ],
        out_specs=[
            pl.BlockSpec((_UVB, H), lambda i: (i, 0)),
            pl.BlockSpec((_UVB, H), lambda i: (i, 0)),
        ],
        out_shape=[
            jax.ShapeDtypeStruct((NP, H), jnp.float32),
            jax.ShapeDtypeStruct((NP, H), jnp.float32),
        ],
    )(xp, Wuv, b1r)


_NC = 2
_NS = 16
_NW = _NC * _NS
_PT = EP // _NW
_CH = 320
_NCH = _PT // _CH


def _sc_gather(v_tab, idx_flat):
    mesh = plsc.VectorSubcoreMesh(core_axis_name="c", subcore_axis_name="s")

    @functools.partial(
        pl.kernel,
        out_type=jax.ShapeDtypeStruct((EP, H), jnp.float32),
        mesh=mesh,
        scratch_types=[
            pltpu.VMEM((_PT,), jnp.int32),
            pltpu.VMEM((_CH, H), jnp.float32),
            pltpu.VMEM((_CH, H), jnp.float32),
            pltpu.SemaphoreType.DMA,
        ],
    )
    def body(v_hbm, idx_hbm, out_hbm, idx_v, buf0, buf1, sem):
        wid = lax.axis_index("s") * _NC + lax.axis_index("c")
        base = wid * _PT
        pltpu.sync_copy(idx_hbm.at[pl.ds(base, _PT)], idx_v)
        bufs = (buf0, buf1)
        pend = pltpu.async_copy(v_hbm.at[idx_v.at[pl.ds(0, _CH)]], buf0, sem)
        for c in range(_NCH):
            pend.wait()
            if c + 1 < _NCH:
                pend = pltpu.async_copy(
                    v_hbm.at[idx_v.at[pl.ds((c + 1) * _CH, _CH)]],
                    bufs[(c + 1) % 2], sem)
            pltpu.sync_copy(bufs[c % 2],
                            out_hbm.at[pl.ds(base + c * _CH, _CH)])

    return body(v_tab, idx_flat)


def _urep(u_blk):
    return jnp.broadcast_to(u_blk[:, None, :], (UB, K, H)).reshape(EB, H)


def _stats1_body(vg_ref, u_ref, out_ref, acc_ref):
    b = pl.program_id(0)

    @pl.when(b == 0)
    def _():
        acc_ref[...] = jnp.zeros((8, H), jnp.float32)

    g = vg_ref[...] + _urep(u_ref[...])
    s = jnp.sum(g, axis=0, keepdims=True)
    ss = jnp.sum(g * g, axis=0, keepdims=True)
    acc_ref[0:1, :] += s
    acc_ref[1:2, :] += ss

    @pl.when(b == pl.num_programs(0) - 1)
    def _():
        out_ref[...] = acc_ref[...]


def _stats1(vg, u):
    return pl.pallas_call(
        _stats1_body,
        grid=(EP // EB,),
        in_specs=[
            pl.BlockSpec((EB, H), lambda i: (i, 0)),
            pl.BlockSpec((UB, H), lambda i: (i, 0)),
        ],
        out_specs=pl.BlockSpec((8, H), lambda i: (0, 0)),
        out_shape=jax.ShapeDtypeStruct((8, H), jnp.float32),
        scratch_shapes=[pltpu.VMEM((8, H), jnp.float32)],
    )(vg, u)


def _layer2_body(vg_ref, u_ref, sc1_ref, sh1_ref, w2_ref, b2_ref,
                 h2p_ref, out_ref, acc_ref):
    b = pl.program_id(0)

    @pl.when(b == 0)
    def _():
        acc_ref[...] = jnp.zeros((8, H), jnp.float32)

    g = vg_ref[...] + _urep(u_ref[...])
    h1 = jnp.maximum(g * sc1_ref[...] + sh1_ref[...], 0.0)
    e_g = b * EB + lax.broadcasted_iota(jnp.int32, (EB, 1), 0)
    mask = e_g < N * K
    h1 = jnp.where(mask, h1, 0.0)
    h2 = jnp.dot(h1, w2_ref[...], preferred_element_type=jnp.float32,
                 precision=_HI)
    h2 = h2 + jnp.where(mask, b2_ref[...], 0.0)
    h2p_ref[...] = h2
    acc_ref[0:1, :] += jnp.sum(h2, axis=0, keepdims=True)
    acc_ref[1:2, :] += jnp.sum(h2 * h2, axis=0, keepdims=True)

    @pl.when(b == pl.num_programs(0) - 1)
    def _():
        out_ref[...] = acc_ref[...]


def _layer2(vg, u, sc1, sh1, W2, b2r):
    return pl.pallas_call(
        _layer2_body,
        grid=(EP // EB,),
        in_specs=[
            pl.BlockSpec((EB, H), lambda i: (i, 0)),
            pl.BlockSpec((UB, H), lambda i: (i, 0)),
            pl.BlockSpec((1, H), lambda i: (0, 0)),
            pl.BlockSpec((1, H), lambda i: (0, 0)),
            pl.BlockSpec((H, H), lambda i: (0, 0)),
            pl.BlockSpec((1, H), lambda i: (0, 0)),
        ],
        out_specs=[
            pl.BlockSpec((EB, H), lambda i: (i, 0)),
            pl.BlockSpec((8, H), lambda i: (0, 0)),
        ],
        out_shape=[
            jax.ShapeDtypeStruct((EP, H), jnp.float32),
            jax.ShapeDtypeStruct((8, H), jnp.float32),
        ],
        scratch_shapes=[pltpu.VMEM((8, H), jnp.float32)],
    )(vg, u, sc1, sh1, W2, b2r)


def _reduce_body(h2p_ref, sc2_ref, sh2_ref, out_ref):
    h2 = jnp.maximum(h2p_ref[...] * sc2_ref[...] + sh2_ref[...], 0.0)
    r3 = h2.reshape(UB, K, H)
    acc = r3[:, 0, :]
    for t in range(1, K):
        acc = acc + r3[:, t, :]
    out_ref[...] = acc * (1.0 / K)


def _reduce(h2p, sc2, sh2):
    return pl.pallas_call(
        _reduce_body,
        grid=(EP // EB,),
        in_specs=[
            pl.BlockSpec((EB, H), lambda i: (i, 0)),
            pl.BlockSpec((1, H), lambda i: (0, 0)),
            pl.BlockSpec((1, H), lambda i: (0, 0)),
        ],
        out_specs=pl.BlockSpec((UB, H), lambda i: (i, 0)),
        out_shape=jax.ShapeDtypeStruct((NP, H), jnp.float32),
    )(h2p, sc2, sh2)


@jax.jit
def kernel(x, W1, b1, g1, be1, W2, b2, g2, be2):
    xp = jnp.zeros((NP, D), jnp.float32).at[:N].set(x)
    pos_pad = jnp.full((NP, 2), 1e18, jnp.float32).at[:N].set(x[:, :2])
    posT = pos_pad.T

    idx = _knn(pos_pad, posT)

    Wuv = jnp.concatenate([W1[:D] - W1[D:], W1[D:]], axis=1)
    u, v = _uv(xp, Wuv, b1[None])

    vg = _sc_gather(v, idx.reshape(-1))

    st1 = _stats1(vg, u)
    m1 = st1[0] / E_REAL
    var1 = jnp.maximum(st1[1] / E_REAL - m1 * m1, 0.0)
    sc1 = g1 / jnp.sqrt(var1 + 1e-5)
    sh1 = be1 - m1 * sc1

    h2p, st2 = _layer2(vg, u, sc1[None], sh1[None], W2, b2[None])
    m2 = st2[0] / E_REAL
    var2 = jnp.maximum(st2[1] / E_REAL - m2 * m2, 0.0)
    sc2 = g2 / jnp.sqrt(var2 + 1e-5)
    sh2 = be2 - m2 * sc2

    outp = _reduce(h2p, sc2[None], sh2[None])
    return outp[:N]
